# baseline (speedup 1.0000x reference)
import functools

import jax
import jax.numpy as jnp
from jax import lax
from jax.experimental import pallas as pl
from jax.experimental.pallas import tpu as pltpu
from jax.experimental.pallas import tpu_sc as plsc

_S = 7.0
_CELLS = 6272
_NW = 32
_CPW = _CELLS // _NW
_WORDS = _CPW * 30
_STEPS = 13


def _isqrt_prod(a, b):
    x = a * b
    xc = jnp.maximum(x, 1e-30)
    i = lax.bitcast_convert_type(xc, jnp.int32)
    i = 0x5F3759DF - (i >> 1)
    y = lax.bitcast_convert_type(i, jnp.float32)
    for _ in range(3):
        y = y * (1.5 - 0.5 * xc * y * y)
    return x * y


def _sc_body(p_hbm, g_hbm, out_hbm, p_v, g_v, acc_v):
    wid = lax.axis_index("s") * 2 + lax.axis_index("c")
    base = wid * _WORDS
    pltpu.sync_copy(p_hbm.at[pl.ds(base, _WORDS)], p_v)
    pltpu.sync_copy(g_hbm.at[pl.ds(base, _WORDS)], g_v)

    lanes = lax.iota(jnp.int32, 16)
    zero = jnp.zeros((16,), jnp.float32)
    acc = zero

    for j in range(_STEPS):
        rows = j * 16 + lanes
        valid = rows < _CPW
        off = jnp.minimum(rows, _CPW - 1) * 30

        def pch(c):
            return plsc.load_gather(p_v, [off + c])

        def gch(c):
            return plsc.load_gather(g_v, [off + c])

        g4 = gch(4)
        obj = g4 > 0.0

        d4 = pch(4) - g4
        d9 = pch(9) - gch(9)
        noobj_t = d4 * d4 + d9 * d9

        class_t = zero
        for c in range(10, 30):
            dc = pch(c) - gch(c)
            class_t = class_t + dc * dc

        gx, gy, gw, gh = gch(0), gch(1), gch(2), gch(3)
        gcx = gx * (1.0 / _S)
        gcy = gy * (1.0 / _S)
        tx1 = gcx - gw * 0.5
        ty1 = gcy - gh * 0.5
        tx2 = gcx + gw * 0.5
        ty2 = gcy + gh * 0.5
        area_t = (tx2 - tx1) * (ty2 - ty1)

        def iou_of(x, y, w, h):
            cx = x * (1.0 / _S)
            cy = y * (1.0 / _S)
            x1 = cx - w * 0.5
            y1 = cy - h * 0.5
            x2 = cx + w * 0.5
            y2 = cy + h * 0.5
            iw = jnp.maximum(
                jnp.minimum(x2, tx2) - jnp.maximum(x1, tx1), 0.0)
            ih = jnp.maximum(
                jnp.minimum(y2, ty2) - jnp.maximum(y1, ty1), 0.0)
            inter = iw * ih
            area_p = (x2 - x1) * (y2 - y1)
            return inter / (area_p + area_t - inter)

        p0x, p0y, p0w, p0h, p0c = pch(0), pch(1), pch(2), pch(3), pch(4)
        p1x, p1y, p1w, p1h, p1c = pch(5), pch(6), pch(7), pch(8), pch(9)
        iou0 = iou_of(p0x, p0y, p0w, p0h)
        iou1 = iou_of(p1x, p1y, p1w, p1h)
        sel = iou1 > iou0
        max_iou = jnp.maximum(iou0, iou1)

        prx = jnp.where(sel, p1x, p0x)
        pry = jnp.where(sel, p1y, p0y)
        prw = jnp.where(sel, p1w, p0w)
        prh = jnp.where(sel, p1h, p0h)
        prc = jnp.where(sel, p1c, p0c)
        trx = jnp.where(sel, gch(5), gx)
        try_ = jnp.where(sel, gch(6), gy)
        trw = jnp.where(sel, gch(7), gw)
        trh = jnp.where(sel, gch(8), gh)

        dx = prx - trx
        dy = pry - try_
        xy_t = dx * dx + dy * dy
        wh_t = (prw + trw - 2.0 * _isqrt_prod(prw, trw)
                + prh + trh - 2.0 * _isqrt_prod(prh, trh))
        do = prc - max_iou
        obj_t = do * do

        cell = jnp.where(
            obj,
            5.0 * (xy_t + wh_t) + obj_t + class_t,
            0.5 * noobj_t,
        )
        acc = acc + jnp.where(valid, cell, 0.0)

    acc_v[...] = acc
    pltpu.sync_copy(acc_v, out_hbm.at[wid])


@functools.partial(
    pl.kernel,
    mesh=plsc.VectorSubcoreMesh(core_axis_name="c", subcore_axis_name="s"),
    compiler_params=pltpu.CompilerParams(needs_layout_passes=False),
    out_type=jax.ShapeDtypeStruct((_NW, 16), jnp.float32),
    scratch_types=[
        pltpu.VMEM((_WORDS,), jnp.float32),
        pltpu.VMEM((_WORDS,), jnp.float32),
        pltpu.VMEM((16,), jnp.float32),
    ],
)
def _sc_loss(p_hbm, g_hbm, out_hbm, p_v, g_v, acc_v):
    _sc_body(p_hbm, g_hbm, out_hbm, p_v, g_v, acc_v)


def kernel(prediction, gt_tensor):
    p = prediction.reshape(_CELLS * 30)
    g = gt_tensor.reshape(_CELLS * 30)
    partials = _sc_loss(p, g)
    return jnp.sum(partials) * (1.0 / 128.0)


# speedup vs baseline: 1.0174x; 1.0174x over previous
import functools

import jax
import jax.numpy as jnp
from jax import lax
from jax.experimental import pallas as pl
from jax.experimental.pallas import tpu as pltpu
from jax.experimental.pallas import tpu_sc as plsc

_S = 7.0
_CELLS = 6272
_NW = 32
_CPW = _CELLS // _NW
_WORDS = _CPW * 30
_STEPS = 13


def _isqrt_prod(a, b):
    x = a * b
    xc = jnp.maximum(x, 1e-30)
    i = lax.bitcast_convert_type(xc, jnp.int32)
    i = 0x5F3759DF - (i >> 1)
    y = lax.bitcast_convert_type(i, jnp.float32)
    for _ in range(3):
        y = y * (1.5 - 0.5 * xc * y * y)
    return x * y


def _sc_body(p_hbm, g_hbm, out_hbm, p_v, g_v, acc_v, sem_p, sem_g):
    wid = lax.axis_index("s") * 2 + lax.axis_index("c")
    base = wid * _WORDS
    cp_p = pltpu.async_copy(p_hbm.at[pl.ds(base, _WORDS)], p_v, sem_p)
    cp_g = pltpu.async_copy(g_hbm.at[pl.ds(base, _WORDS)], g_v, sem_g)
    cp_p.wait()
    cp_g.wait()

    lanes = lax.iota(jnp.int32, 16)
    zero = jnp.zeros((16,), jnp.float32)
    acc = zero

    for j in range(_STEPS):
        rows = j * 16 + lanes
        valid = rows < _CPW
        off = jnp.minimum(rows, _CPW - 1) * 30

        def pch(c):
            return plsc.load_gather(p_v, [off + c])

        def gch(c):
            return plsc.load_gather(g_v, [off + c])

        g4 = gch(4)
        obj = g4 > 0.0

        d4 = pch(4) - g4
        d9 = pch(9) - gch(9)
        noobj_t = d4 * d4 + d9 * d9

        class_t = zero
        for c in range(10, 30):
            dc = pch(c) - gch(c)
            class_t = class_t + dc * dc

        gx, gy, gw, gh = gch(0), gch(1), gch(2), gch(3)
        gcx = gx * (1.0 / _S)
        gcy = gy * (1.0 / _S)
        tx1 = gcx - gw * 0.5
        ty1 = gcy - gh * 0.5
        tx2 = gcx + gw * 0.5
        ty2 = gcy + gh * 0.5
        area_t = (tx2 - tx1) * (ty2 - ty1)

        def iou_of(x, y, w, h):
            cx = x * (1.0 / _S)
            cy = y * (1.0 / _S)
            x1 = cx - w * 0.5
            y1 = cy - h * 0.5
            x2 = cx + w * 0.5
            y2 = cy + h * 0.5
            iw = jnp.maximum(
                jnp.minimum(x2, tx2) - jnp.maximum(x1, tx1), 0.0)
            ih = jnp.maximum(
                jnp.minimum(y2, ty2) - jnp.maximum(y1, ty1), 0.0)
            inter = iw * ih
            area_p = (x2 - x1) * (y2 - y1)
            return inter / (area_p + area_t - inter)

        p0x, p0y, p0w, p0h, p0c = pch(0), pch(1), pch(2), pch(3), pch(4)
        p1x, p1y, p1w, p1h, p1c = pch(5), pch(6), pch(7), pch(8), pch(9)
        iou0 = iou_of(p0x, p0y, p0w, p0h)
        iou1 = iou_of(p1x, p1y, p1w, p1h)
        sel = iou1 > iou0
        max_iou = jnp.maximum(iou0, iou1)

        prx = jnp.where(sel, p1x, p0x)
        pry = jnp.where(sel, p1y, p0y)
        prw = jnp.where(sel, p1w, p0w)
        prh = jnp.where(sel, p1h, p0h)
        prc = jnp.where(sel, p1c, p0c)
        trx = jnp.where(sel, gch(5), gx)
        try_ = jnp.where(sel, gch(6), gy)
        trw = jnp.where(sel, gch(7), gw)
        trh = jnp.where(sel, gch(8), gh)

        dx = prx - trx
        dy = pry - try_
        xy_t = dx * dx + dy * dy
        wh_t = (prw + trw - 2.0 * _isqrt_prod(prw, trw)
                + prh + trh - 2.0 * _isqrt_prod(prh, trh))
        do = prc - max_iou
        obj_t = do * do

        cell = jnp.where(
            obj,
            5.0 * (xy_t + wh_t) + obj_t + class_t,
            0.5 * noobj_t,
        )
        acc = acc + jnp.where(valid, cell, 0.0)

    acc_v[...] = acc
    pltpu.sync_copy(acc_v, out_hbm.at[wid])


@functools.partial(
    pl.kernel,
    mesh=plsc.VectorSubcoreMesh(core_axis_name="c", subcore_axis_name="s"),
    compiler_params=pltpu.CompilerParams(
        needs_layout_passes=False,
        skip_device_barrier=True,
        disable_bounds_checks=True,
        disable_semaphore_checks=True,
    ),
    out_type=jax.ShapeDtypeStruct((_NW, 16), jnp.float32),
    scratch_types=[
        pltpu.VMEM((_WORDS,), jnp.float32),
        pltpu.VMEM((_WORDS,), jnp.float32),
        pltpu.VMEM((16,), jnp.float32),
        pltpu.SemaphoreType.DMA,
        pltpu.SemaphoreType.DMA,
    ],
)
def _sc_loss(p_hbm, g_hbm, out_hbm, p_v, g_v, acc_v, sem_p, sem_g):
    _sc_body(p_hbm, g_hbm, out_hbm, p_v, g_v, acc_v, sem_p, sem_g)


def kernel(prediction, gt_tensor):
    p = prediction.reshape(_CELLS * 30)
    g = gt_tensor.reshape(_CELLS * 30)
    partials = _sc_loss(p, g)
    return jnp.sum(partials) * (1.0 / 128.0)
